# baseline (speedup 1.0000x reference)
import functools

import jax
import jax.numpy as jnp
from jax import lax
from jax.experimental import pallas as pl
from jax.experimental.pallas import tpu as pltpu
from jax.experimental.pallas import tpu_sc as plsc

_NUM_CORES = 2
_NUM_SUBCORES = 16
_LANES = 16


@functools.cache
def _fused_call(n, d, e):
    nw = _NUM_CORES * _NUM_SUBCORES
    lanes = _LANES
    assert n == 10000 and d == 128 and e % (nw * lanes) == 0, (n, d, e)
    e_per_w = e // nw
    nvec = e_per_w // lanes

    rpt = n // _NUM_SUBCORES
    rpt_pad = 640
    n_pad = _NUM_SUBCORES * rpt_pad
    chunk_rows = 125
    n_chunks = rpt // chunk_rows
    groups = rpt_pad // lanes

    mesh = plsc.VectorSubcoreMesh(
        core_axis_name="c", subcore_axis_name="s",
        num_cores=_NUM_CORES, num_subcores=_NUM_SUBCORES,
    )

    @functools.partial(
        pl.kernel,
        out_type=(
            jax.ShapeDtypeStruct((2 * e,), jnp.int32),
            jax.ShapeDtypeStruct((e,), jnp.float32),
        ),
        mesh=mesh,
        compiler_params=pltpu.CompilerParams(needs_layout_passes=False),
        scratch_types=[
            pltpu.VMEM((chunk_rows * d,), jnp.float32),
            pltpu.VMEM((chunk_rows * d,), jnp.float32),
            pltpu.VMEM((d,), jnp.float32),
            pltpu.VMEM((lanes,), jnp.float32),
            pltpu.VMEM((rpt_pad * lanes,), jnp.float32),
            pltpu.VMEM((rpt_pad,), jnp.float32),
            pltpu.VMEM((n_pad,), jnp.float32),
            pltpu.VMEM_SHARED((n_pad,), jnp.float32),
            pltpu.VMEM((e_per_w,), jnp.int32),
            pltpu.VMEM((e_per_w,), jnp.int32),
            pltpu.VMEM((e_per_w,), jnp.float32),
            pltpu.VMEM((e_per_w,), jnp.float32),
            pltpu.SemaphoreType.DMA,
            pltpu.SemaphoreType.DMA,
            pltpu.SemaphoreType.DMA,
        ],
    )
    def fused_k(x_hbm, p_hbm, b_hbm, ei_hbm, attr_hbm, ei_out, val_out,
                xbuf0, xbuf1, p_v, b_v, acc_v, sig_v, diag_v, shared_diag,
                idx_v, src_v, attr_v, val_v, sem_edge, sem_x, sem_out):
        cid = lax.axis_index("c")
        sid = lax.axis_index("s")
        wid = sid * _NUM_CORES + cid
        ebase = wid * e_per_w
        sl = pl.ds(ebase, e_per_w)
        sl_row0 = pl.ds(ebase, e_per_w)
        sl_row1 = pl.ds(e + ebase, e_per_w)

        cp_idx = pltpu.make_async_copy(ei_hbm.at[sl_row1], idx_v, sem_edge)
        cp_src = pltpu.make_async_copy(ei_hbm.at[sl_row0], src_v, sem_edge)
        cp_attr = pltpu.make_async_copy(attr_hbm.at[sl], attr_v, sem_edge)
        cp_idx.start()
        cp_src.start()
        cp_attr.start()

        pltpu.sync_copy(p_hbm, p_v)
        pltpu.sync_copy(b_hbm, b_v)
        pvecs = [p_v[pl.ds(j * lanes, lanes)] for j in range(d // lanes)]
        b_vec = b_v[...]

        xbufs = [xbuf0, xbuf1]
        row0_flat = (sid * rpt) * d
        cp_cur = pltpu.make_async_copy(
            x_hbm.at[pl.ds(row0_flat, chunk_rows * d)], xbuf0, sem_x)
        cp_cur.start()
        for ci in range(n_chunks):
            if ci + 1 < n_chunks:
                cp_next = pltpu.make_async_copy(
                    x_hbm.at[pl.ds(row0_flat + (ci + 1) * chunk_rows * d,
                                   chunk_rows * d)],
                    xbufs[(ci + 1) % 2], sem_x)
                cp_next.start()
            cp_cur.wait()
            xchunk = xbufs[ci % 2]
            acc_base = ci * chunk_rows * lanes

            def row_body(r, carry, xchunk=xchunk, acc_base=acc_base):
                off = r * d
                acc = xchunk[pl.ds(off, lanes)] * pvecs[0]
                for j in range(1, d // lanes):
                    acc = acc + xchunk[pl.ds(off + j * lanes, lanes)] * pvecs[j]
                acc_v[pl.ds(acc_base + r * lanes, lanes)] = acc
                return carry

            lax.fori_loop(0, chunk_rows, row_body, 0, unroll=4)
            if ci + 1 < n_chunks:
                cp_cur = cp_next

        zeros = jnp.zeros((lanes,), jnp.float32)
        for r in range(rpt, rpt_pad):
            acc_v[pl.ds(r * lanes, lanes)] = zeros

        stride_iota = lax.iota(jnp.int32, lanes) * lanes
        for g in range(groups):
            gbase = g * lanes * lanes
            s = plsc.load_gather(acc_v, [stride_iota + gbase])
            for l in range(1, lanes):
                s = s + plsc.load_gather(acc_v, [stride_iota + (gbase + l)])
            z = s + b_vec
            sig_v[pl.ds(g * lanes, lanes)] = 1.0 / (1.0 + jnp.exp(-z))

        pltpu.sync_copy(sig_v, shared_diag.at[pl.ds(sid * rpt_pad, rpt_pad)])
        plsc.subcore_barrier()
        pltpu.sync_copy(shared_diag, diag_v)

        cp_idx.wait()
        cp_src.wait()
        cp_attr.wait()
        cp_ei0 = pltpu.make_async_copy(src_v, ei_out.at[sl_row0], sem_out)
        cp_ei1 = pltpu.make_async_copy(idx_v, ei_out.at[sl_row1], sem_out)
        cp_ei0.start()
        cp_ei1.start()

        def body(i, carry):
            s = pl.ds(i * lanes, lanes)
            nidx = idx_v[s]
            t = lax.shift_right_logical(nidx * 26844, 24)
            loc = nidx + t * 15
            vals = plsc.load_gather(diag_v, [loc])
            val_v[s] = vals * attr_v[s]
            return carry

        lax.fori_loop(0, nvec, body, 0, unroll=8)

        cp_val = pltpu.make_async_copy(val_v, val_out.at[sl], sem_out)
        cp_val.start()
        cp_ei0.wait()
        cp_ei1.wait()
        cp_val.wait()

    return fused_k


def kernel(x, edge_index, edge_attr, p, b):
    n, d = x.shape
    e = edge_attr.shape[0]
    ei_flat, adj_val = _fused_call(n, d, e)(
        x.reshape(n * d),
        p.reshape(d),
        jnp.broadcast_to(b, (_LANES,)),
        edge_index.reshape(2 * e),
        edge_attr,
    )
    return (ei_flat.reshape(2, e), adj_val)


# speedup vs baseline: 1.2270x; 1.2270x over previous
import functools

import jax
import jax.numpy as jnp
from jax import lax
from jax.experimental import pallas as pl
from jax.experimental.pallas import tpu as pltpu
from jax.experimental.pallas import tpu_sc as plsc


def _diag_body(x_ref, p_ref, b_ref, out_ref):
    z = jnp.dot(x_ref[...], p_ref[...], preferred_element_type=jnp.float32)
    out_ref[...] = jax.nn.sigmoid(z + b_ref[...]).T


@functools.cache
def _diag_call(n, d):
    return pl.pallas_call(
        _diag_body,
        out_shape=jax.ShapeDtypeStruct((1, n), jnp.float32),
    )


_NUM_CORES = 2
_NUM_SUBCORES = 16
_LANES = 16


@functools.cache
def _gather_call(n, e):
    nw = _NUM_CORES * _NUM_SUBCORES
    lanes = _LANES
    assert e % (nw * lanes) == 0, (e, nw, lanes)
    e_per_w = e // nw
    nvec = e_per_w // lanes
    mesh = plsc.VectorSubcoreMesh(
        core_axis_name="c", subcore_axis_name="s",
        num_cores=_NUM_CORES, num_subcores=_NUM_SUBCORES,
    )

    @functools.partial(
        pl.kernel,
        out_type=(
            jax.ShapeDtypeStruct((2 * e,), jnp.int32),
            jax.ShapeDtypeStruct((e,), jnp.float32),
        ),
        mesh=mesh,
        compiler_params=pltpu.CompilerParams(needs_layout_passes=False),
        scratch_types=[
            pltpu.VMEM((n,), jnp.float32),
            pltpu.VMEM((e_per_w,), jnp.int32),
            pltpu.VMEM((e_per_w,), jnp.int32),
            pltpu.VMEM((e_per_w,), jnp.float32),
            pltpu.VMEM((e_per_w,), jnp.float32),
            pltpu.SemaphoreType.DMA,
            pltpu.SemaphoreType.DMA,
        ],
    )
    def gather_k(diag_hbm, ei_hbm, attr_hbm, ei_out, val_out,
                 diag_v, idx_v, src_v, attr_v, val_v, sem_in, sem_out):
        wid = lax.axis_index("s") * _NUM_CORES + lax.axis_index("c")
        base = wid * e_per_w
        sl = pl.ds(base, e_per_w)

        sl_row0 = pl.ds(base, e_per_w)
        sl_row1 = pl.ds(e + base, e_per_w)

        cp_diag = pltpu.make_async_copy(diag_hbm, diag_v, sem_in)
        cp_idx = pltpu.make_async_copy(ei_hbm.at[sl_row1], idx_v, sem_in)
        cp_attr = pltpu.make_async_copy(attr_hbm.at[sl], attr_v, sem_in)
        cp_src = pltpu.make_async_copy(ei_hbm.at[sl_row0], src_v, sem_in)
        cp_diag.start()
        cp_idx.start()
        cp_attr.start()
        cp_src.start()
        cp_diag.wait()
        cp_idx.wait()
        cp_attr.wait()
        cp_src.wait()
        cp_ei0 = pltpu.make_async_copy(src_v, ei_out.at[sl_row0], sem_out)
        cp_ei1 = pltpu.make_async_copy(idx_v, ei_out.at[sl_row1], sem_out)
        cp_ei0.start()
        cp_ei1.start()

        def body(i, carry):
            s = pl.ds(i * lanes, lanes)
            idx = idx_v[s]
            vals = plsc.load_gather(diag_v, [idx])
            val_v[s] = vals * attr_v[s]
            return carry

        lax.fori_loop(0, nvec, body, 0, unroll=8)

        cp_val = pltpu.make_async_copy(val_v, val_out.at[sl], sem_out)
        cp_val.start()
        cp_ei0.wait()
        cp_ei1.wait()
        cp_val.wait()

    return gather_k


def kernel(x, edge_index, edge_attr, p, b):
    n, d = x.shape
    e = edge_attr.shape[0]
    diag = _diag_call(n, d)(x, p, b.reshape(1, 1)).reshape(n)
    ei_flat, adj_val = _gather_call(n, e)(diag, edge_index.reshape(2 * e), edge_attr)
    return (ei_flat.reshape(2, e), adj_val)
